# baseline (speedup 1.0000x reference)
import jax
import jax.numpy as jnp
from jax.experimental import pallas as pl
from jax.experimental.pallas import tpu as pltpu


def _vq_kernel(z_ref, cb_ref, out_ref):
    z4 = z_ref[0]
    z = z4.reshape(256, 1024)
    in_sqr = jnp.sum(z * z, axis=0, keepdims=True)
    out_ref[...] = in_sqr.astype(jnp.int32).reshape(1, 1, -1)


def kernel(z_e_x, codebook):
    B, D, H, W = z_e_x.shape
    K = codebook.shape[0]
    NL = H * W
    out = pl.pallas_call(
        _vq_kernel,
        grid=(B,),
        in_specs=[
            pl.BlockSpec((1, D, H, W), lambda i: (i, 0, 0, 0)),
            pl.BlockSpec((K, D), lambda i: (0, 0)),
        ],
        out_specs=pl.BlockSpec((1, 1, NL), lambda i: (i, 0, 0)),
        out_shape=jax.ShapeDtypeStruct((B, 1, NL), jnp.int32),
    )(z_e_x, codebook)
    return out.reshape(B, H, W)


# speedup vs baseline: 1.8971x; 1.8971x over previous
import jax
import jax.numpy as jnp
from jax.experimental import pallas as pl
from jax.experimental.pallas import tpu as pltpu


def _vq_kernel(z_ref, cb_ref, out_ref, cbbf_scr, cbs_scr):
    @pl.when(pl.program_id(0) == 0)
    def _prep():
        cb = cb_ref[...]
        clo = cb[:, :128]
        chi = cb[:, 128:]
        cbs_scr[...] = jnp.sum(clo * clo + chi * chi, axis=1, keepdims=True)
        cbbf_scr[...] = cb.astype(jnp.bfloat16)

    z = z_ref[0]
    K = cbbf_scr.shape[0]
    in_sqr = jnp.sum(z * z, axis=0, keepdims=True)
    mm2 = jax.lax.dot_general(
        cbbf_scr[...], (z + z).astype(jnp.bfloat16),
        (((1,), (0,)), ((), ())),
        preferred_element_type=jnp.float32)
    dist = (cbs_scr[...] + in_sqr) - mm2
    minv = jnp.min(dist, axis=0, keepdims=True)
    kv = jax.lax.broadcasted_iota(jnp.int32, dist.shape, 0)
    cand = jnp.where(dist == minv, kv, jnp.int32(K))
    out_ref[...] = jnp.min(cand, axis=0).reshape(1, 1, -1)


def kernel(z_e_x, codebook):
    B, D, H, W = z_e_x.shape
    K = codebook.shape[0]
    NL = H * W
    z3 = z_e_x.reshape(B, D, NL)
    out = pl.pallas_call(
        _vq_kernel,
        grid=(B,),
        in_specs=[
            pl.BlockSpec((1, D, NL), lambda i: (i, 0, 0)),
            pl.BlockSpec((K, D), lambda i: (0, 0)),
        ],
        out_specs=pl.BlockSpec((1, 1, NL), lambda i: (i, 0, 0)),
        out_shape=jax.ShapeDtypeStruct((B, 1, NL), jnp.int32),
        scratch_shapes=[
            pltpu.VMEM((K, D), jnp.bfloat16),
            pltpu.VMEM((K, 1), jnp.float32),
        ],
    )(z3, codebook)
    return out.reshape(B, H, W)


# speedup vs baseline: 2.8812x; 1.5187x over previous
import jax
import jax.numpy as jnp
from jax.experimental import pallas as pl
from jax.experimental.pallas import tpu as pltpu


def _vq_kernel(x_ref, cb_ref, out_ref, cbbf_scr, cbs_scr):
    @pl.when(pl.program_id(0) == 0)
    def _prep():
        cb = cb_ref[...]
        clo = cb[:, :128]
        chi = cb[:, 128:]
        cbs_scr[...] = jnp.sum(clo * clo + chi * chi, axis=1, keepdims=True)
        cbbf_scr[...] = cb.astype(jnp.bfloat16)

    x = x_ref[...]
    K = cbbf_scr.shape[0]
    xlo = x[:, :128]
    xhi = x[:, 128:]
    in_sqr = jnp.sum(xlo * xlo + xhi * xhi, axis=1, keepdims=True)
    mm2 = jax.lax.dot_general(
        cbbf_scr[...], (x + x).astype(jnp.bfloat16),
        (((1,), (1,)), ((), ())),
        preferred_element_type=jnp.float32)
    dist = (cbs_scr[...] + in_sqr.T) - mm2
    minv = jnp.min(dist, axis=0, keepdims=True)
    kv = jax.lax.broadcasted_iota(jnp.int32, dist.shape, 0)
    cand = jnp.where(dist == minv, kv, jnp.int32(K))
    out_ref[...] = jnp.min(cand, axis=0).reshape(1, 1, -1)


def kernel(z_e_x, codebook):
    B, D, H, W = z_e_x.shape
    K = codebook.shape[0]
    flat = jnp.transpose(z_e_x, (0, 2, 3, 1)).reshape(-1, D)
    N = flat.shape[0]
    BN = 2048
    NB = N // BN
    out = pl.pallas_call(
        _vq_kernel,
        grid=(NB,),
        in_specs=[
            pl.BlockSpec((BN, D), lambda i: (i, 0)),
            pl.BlockSpec((K, D), lambda i: (0, 0)),
        ],
        out_specs=pl.BlockSpec((1, 1, BN), lambda i: (i, 0, 0)),
        out_shape=jax.ShapeDtypeStruct((NB, 1, BN), jnp.int32),
        scratch_shapes=[
            pltpu.VMEM((K, D), jnp.bfloat16),
            pltpu.VMEM((K, 1), jnp.float32),
        ],
    )(flat, codebook)
    return out.reshape(B, H, W)


# speedup vs baseline: 2.8899x; 1.0030x over previous
import jax
import jax.numpy as jnp
from jax.experimental import pallas as pl
from jax.experimental.pallas import tpu as pltpu


def _vq_kernel(x_ref, cb_ref, out_ref, cbbf_scr, cbs_scr):
    @pl.when(pl.program_id(0) == 0)
    def _prep():
        cb = cb_ref[...]
        clo = cb[:, :128]
        chi = cb[:, 128:]
        cbs_scr[...] = jnp.sum(clo * clo + chi * chi, axis=1, keepdims=True)
        cbbf_scr[...] = (cb + cb).astype(jnp.bfloat16)

    x = x_ref[...]
    K = cbbf_scr.shape[0]
    xlo = x[:, :128]
    xhi = x[:, 128:]
    in_sqr = jnp.sum(xlo * xlo + xhi * xhi, axis=1, keepdims=True)
    mm2 = jax.lax.dot_general(
        cbbf_scr[...], x.astype(jnp.bfloat16),
        (((1,), (1,)), ((), ())),
        preferred_element_type=jnp.float32)
    dist = (cbs_scr[...] + in_sqr.T) - mm2
    minv = jnp.min(dist, axis=0, keepdims=True)
    kv = jax.lax.broadcasted_iota(jnp.int32, dist.shape, 0)
    cand = jnp.where(dist == minv, kv, jnp.int32(K))
    out_ref[...] = jnp.min(cand, axis=0).reshape(1, 1, -1)


def kernel(z_e_x, codebook):
    B, D, H, W = z_e_x.shape
    K = codebook.shape[0]
    flat = jnp.transpose(z_e_x, (0, 2, 3, 1)).reshape(-1, D)
    N = flat.shape[0]
    BN = 2048
    NB = N // BN
    out = pl.pallas_call(
        _vq_kernel,
        grid=(NB,),
        in_specs=[
            pl.BlockSpec((BN, D), lambda i: (i, 0)),
            pl.BlockSpec((K, D), lambda i: (0, 0)),
        ],
        out_specs=pl.BlockSpec((1, 1, BN), lambda i: (i, 0, 0)),
        out_shape=jax.ShapeDtypeStruct((NB, 1, BN), jnp.int32),
        scratch_shapes=[
            pltpu.VMEM((K, D), jnp.bfloat16),
            pltpu.VMEM((K, 1), jnp.float32),
        ],
    )(flat, codebook)
    return out.reshape(B, H, W)
